# Initial kernel scaffold; baseline (speedup 1.0000x reference)
import jax
import jax.numpy as jnp
from jax.experimental import pallas as pl


def kernel(x, pos, batch, edge_index, node_attr, x_final_state, pos_final_state, edge_index_final_state, params):
    raise NotImplementedError("write your pallas kernel here")


# speedup vs baseline: 2.2890x; 2.2890x over previous
import functools
import math

import jax
import jax.numpy as jnp
import numpy as np
from jax import lax
from jax.experimental import pallas as pl
from jax.experimental.pallas import tpu as pltpu
from jax.experimental.pallas import tpu_sc as plsc

_N = 10000
_E = 320000
_G = 16
_NC, _NS = 2, 16
_NW = _NC * _NS
_EPW = _E // _NW
_CH = 80
_NCHUNK = _EPW // _CH
_RPT = _N // _NS
_INV_SQRT_NN = 1.0 / math.sqrt(32.0)

_MESH = dict(core_axis_name="c", subcore_axis_name="s",
             num_cores=_NC, num_subcores=_NS)


@functools.partial(
    pl.kernel,
    out_type=jax.ShapeDtypeStruct((_E, 16), jnp.float32),
    mesh=plsc.VectorSubcoreMesh(**_MESH),
    scratch_types=[
        pltpu.VMEM((_CH,), jnp.int32),
        pltpu.VMEM((_CH,), jnp.int32),
        pltpu.VMEM((_CH, 16), jnp.float32),
        pltpu.VMEM((_CH, 16), jnp.float32),
        pltpu.SemaphoreType.DMA,
    ],
    compiler_params=pltpu.CompilerParams(use_tc_tiling_on_sc=False),
)
def _geom(pos_hbm, src_hbm, dst_hbm, vec_hbm, sidx, didx, ps, pd, sem):
    cid = lax.axis_index("c")
    sid = lax.axis_index("s")
    wid = cid * _NS + sid

    def chunk(k, carry):
        off = wid * _EPW + k * _CH
        pltpu.sync_copy(src_hbm.at[pl.ds(off, _CH)], sidx)
        pltpu.sync_copy(dst_hbm.at[pl.ds(off, _CH)], didx)
        pltpu.async_copy(pos_hbm.at[sidx], ps, sem).wait()
        pltpu.async_copy(pos_hbm.at[didx], pd, sem).wait()

        def row(i, c2):
            ps[i, :] = ps[i, :] - pd[i, :]
            return c2

        lax.fori_loop(0, _CH, row, 0)
        pltpu.sync_copy(ps, vec_hbm.at[pl.ds(off, _CH)])
        return carry

    lax.fori_loop(0, _NCHUNK, chunk, 0)


@functools.lru_cache(maxsize=None)
def _make_msg(dp):
    @functools.partial(
        pl.kernel,
        out_type=jax.ShapeDtypeStruct((2 * _N, dp), jnp.float32),
        mesh=plsc.VectorSubcoreMesh(**_MESH),
        scratch_types=[
            pltpu.VMEM((_CH,), jnp.int32),
            pltpu.VMEM((_CH,), jnp.int32),
            pltpu.VMEM((_CH, dp), jnp.float32),
            pltpu.VMEM((_CH, dp), jnp.float32),
            pltpu.VMEM_SHARED((_N, dp), jnp.float32),
            pltpu.SemaphoreType.DMA,
        ],
        compiler_params=pltpu.CompilerParams(use_tc_tiling_on_sc=False),
    )
    def msg(x_hbm, c_hbm, src_hbm, dst_hbm, out_hbm, sidx, didx, xg, cg,
            aggsh, sem):
        cid = lax.axis_index("c")
        sid = lax.axis_index("s")
        wid = cid * _NS + sid
        zv = jnp.zeros((16,), jnp.float32)

        def zrow(i, c2):
            for j in range(dp // 16):
                xg[i, pl.ds(j * 16, 16)] = zv
            return c2

        lax.fori_loop(0, _CH, zrow, 0)
        nfull = _RPT // _CH
        rem = _RPT - nfull * _CH

        def zslab(t, c2):
            pltpu.sync_copy(xg, aggsh.at[pl.ds(sid * _RPT + t * _CH, _CH)])
            return c2

        lax.fori_loop(0, nfull, zslab, 0)
        if rem:
            pltpu.sync_copy(xg.at[pl.ds(0, rem)],
                            aggsh.at[pl.ds(sid * _RPT + nfull * _CH, rem)])
        plsc.subcore_barrier()

        def chunk(k, carry):
            off = wid * _EPW + k * _CH
            pltpu.sync_copy(src_hbm.at[pl.ds(off, _CH)], sidx)
            pltpu.sync_copy(dst_hbm.at[pl.ds(off, _CH)], didx)
            pltpu.async_copy(x_hbm.at[sidx], xg, sem).wait()
            pltpu.sync_copy(c_hbm.at[pl.ds(off, _CH)], cg)

            def mrow(i, c2):
                for j in range(dp // 16):
                    sl = pl.ds(j * 16, 16)
                    xg[i, sl] = xg[i, sl] * cg[i, sl]
                return c2

            lax.fori_loop(0, _CH, mrow, 0)
            pltpu.sync_copy(xg, aggsh.at[didx], add=True)
            return carry

        lax.fori_loop(0, _NCHUNK, chunk, 0)
        plsc.subcore_barrier()
        pltpu.sync_copy(aggsh.at[pl.ds(sid * _RPT, _RPT)],
                        out_hbm.at[pl.ds(cid * _N + sid * _RPT, _RPT)])

    return msg


_BE = 2000
_S3 = math.sqrt(3.0)
_S5 = math.sqrt(5.0)
_S15 = math.sqrt(15.0)
_RB_VALS = np.linspace(0.0, 5.0, 12)[1:-1]
_RB_STEP = float(_RB_VALS[1] - _RB_VALS[0])
_RB_C = 1.14136 * (math.e ** 2)
_SQRT_NB = math.sqrt(10.0)


def _coef_body(dps, *refs):
    vec_ref = refs[0]
    wrefs = refs[1:1 + 4 * len(dps)]
    orefs = refs[1 + 4 * len(dps):]

    v = vec_ref[...]
    x = v[:, 0:1]
    y = v[:, 1:2]
    z = v[:, 2:3]
    r = jnp.sqrt(x * x + y * y + z * z + 1e-9)
    inv = 1.0 / r
    ux, uy, uz = x * inv, y * inv, z * inv
    one = jnp.ones_like(ux)
    zero = jnp.zeros_like(ux)
    sh = jnp.concatenate([
        one,
        _S3 * ux, _S3 * uy, _S3 * uz,
        _S15 * ux * uz, _S15 * ux * uy,
        _S5 * (uy * uy - 0.5 * (ux * ux + uz * uz)),
        _S15 * uy * uz, 0.5 * _S15 * (uz * uz - ux * ux),
        zero, zero, zero, zero, zero, zero, zero,
    ], axis=1)

    kk = lax.broadcasted_iota(jnp.int32, (1, 10), 1).astype(jnp.float32)
    vals = (kk + 1.0) * _RB_STEP
    diff = (r - vals) * (1.0 / _RB_STEP)
    inside = jnp.abs(diff) < 1.0
    denom = jnp.maximum(jnp.where(inside, 1.0 - diff * diff, 1.0), 1e-6)
    emb = jnp.where(inside, _RB_C * jnp.exp(-1.0 / denom), 0.0) * _SQRT_NB
    emb = jnp.concatenate([emb, jnp.zeros_like(v[:, 0:6])], axis=1)

    for l in range(len(dps)):
        w1, b1, w2, wsh = wrefs[4 * l:4 * l + 4]
        h = jnp.dot(emb, w1[...], preferred_element_type=jnp.float32) + b1[...]
        h = h * jax.nn.sigmoid(h)
        c = (jnp.dot(h, w2[...], preferred_element_type=jnp.float32)
             * jnp.dot(sh, wsh[...], preferred_element_type=jnp.float32))
        orefs[l][...] = c


def _coef(vec, wlist, dps):
    flat_w = [w for tup in wlist for w in tup]
    full = lambda shape: pl.BlockSpec(shape, lambda i: (0, 0))
    return pl.pallas_call(
        functools.partial(_coef_body, tuple(dps)),
        grid=(_E // _BE,),
        in_specs=[pl.BlockSpec((_BE, 16), lambda i: (i, 0))]
        + [full(w.shape) for w in flat_w],
        out_specs=[pl.BlockSpec((_BE, dp), lambda i: (i, 0)) for dp in dps],
        out_shape=[jax.ShapeDtypeStruct((_E, dp), jnp.float32) for dp in dps],
    )(vec, *flat_w)


_BN = 2000


def _node_body(last, x_ref, na_ref, agg_ref, wsc_ref, wmsg_ref, o_ref):
    xn = x_ref[...] * na_ref[...]
    sc = jnp.dot(xn, wsc_ref[...], preferred_element_type=jnp.float32)
    a = (agg_ref[0] + agg_ref[1]) * _INV_SQRT_NN
    out = sc + jnp.dot(a, wmsg_ref[...], preferred_element_type=jnp.float32)
    if not last:
        out = out * jax.nn.sigmoid(out)
    o_ref[...] = out


def _node(xp, na, agg2, wsc, wmsg, last, dpo):
    dpi = xp.shape[1]
    agg3 = agg2.reshape(2, _N, dpi)
    return pl.pallas_call(
        functools.partial(_node_body, last),
        grid=(_N // _BN,),
        in_specs=[
            pl.BlockSpec((_BN, dpi), lambda i: (i, 0)),
            pl.BlockSpec((_BN, 1), lambda i: (i, 0)),
            pl.BlockSpec((2, _BN, dpi), lambda i: (0, i, 0)),
            pl.BlockSpec(wsc.shape, lambda i: (0, 0)),
            pl.BlockSpec(wmsg.shape, lambda i: (0, 0)),
        ],
        out_specs=pl.BlockSpec((_BN, dpo), lambda i: (i, 0)),
        out_shape=jax.ShapeDtypeStruct((_N, dpo), jnp.float32),
    )(xp, na, agg3, wsc, wmsg)


def _norm_body(x_ref, b_ref, o_ref):
    xo = x_ref[...]
    bt = b_ref[...]
    ss = jnp.sum(xo * xo, axis=1, keepdims=True)
    gids = lax.broadcasted_iota(jnp.int32, (1, _G), 1)
    oh = (bt == gids).astype(jnp.float32)
    g = jnp.sum(oh * ss, axis=0, keepdims=True)
    f = jnp.sqrt(g + 1e-12)
    fb = jnp.sum(oh * f, axis=1, keepdims=True)
    o_ref[...] = xo / fb


def _norm(xo, bt):
    return pl.pallas_call(
        _norm_body,
        in_specs=[
            pl.BlockSpec((_N, 128), lambda: (0, 0)),
            pl.BlockSpec((_N, 1), lambda: (0, 0)),
        ],
        out_specs=pl.BlockSpec((_N, 128), lambda: (0, 0)),
        out_shape=jax.ShapeDtypeStruct((_N, 128), jnp.float32),
    )(xo, bt)


def _padw(w, r, c):
    return jnp.zeros((r, c), jnp.float32).at[:w.shape[0], :w.shape[1]].set(w)


def _run_net(pos16, xin, na, src, dst, layers):
    vec = _geom(pos16, src, dst)
    dpis = [128, 64, 64, 64]
    dpos = [64, 64, 64, 128]
    wlist = [
        (
            _padw(p['W1'], 16, 64),
            p['b1'].reshape(1, 64),
            _padw(p['W2'], 64, dpis[l]),
            _padw(p['W_sh'], 16, dpis[l]),
        )
        for l, p in enumerate(layers)
    ]
    cs = _coef(vec, wlist, dpis)
    h = xin
    for l, p in enumerate(layers):
        agg2 = _make_msg(dpis[l])(h, cs[l], src, dst)
        wsc = _padw(p['W_sc'], dpis[l], dpos[l])
        wmsg = _padw(p['W_msg'], dpis[l], dpos[l])
        h = _node(h, na, agg2, wsc, wmsg, last=(l == 3), dpo=dpos[l])
    return h


def kernel(x, pos, batch, edge_index, node_attr, x_final_state,
           pos_final_state, edge_index_final_state, params):
    f32 = jnp.float32
    src = edge_index[0].astype(jnp.int32)
    dst = edge_index[1].astype(jnp.int32)
    srcf = edge_index_final_state[0].astype(jnp.int32)
    dstf = edge_index_final_state[1].astype(jnp.int32)
    pos16 = jnp.zeros((_N, 16), f32).at[:, :3].set(pos)
    posf16 = jnp.zeros((_N, 16), f32).at[:, :3].set(pos_final_state)
    post16 = 0.5 * (pos16 + posf16)
    bt = batch.astype(jnp.int32).reshape(_N, 1)

    na = node_attr
    out_i = _norm(_run_net(pos16, x, na, src, dst, params['init']), bt)
    out_f = _norm(_run_net(posf16, x_final_state, na, srcf, dstf,
                           params['final']), bt)
    x_ts = 0.5 * (out_i + out_f)
    out_ts = _norm(_run_net(post16, x_ts, na, src, dst, params['interp']), bt)
    return out_ts


# speedup vs baseline: 3.3785x; 1.4759x over previous
import functools
import math

import jax
import jax.numpy as jnp
import numpy as np
from jax import lax
from jax.experimental import pallas as pl
from jax.experimental.pallas import tpu as pltpu
from jax.experimental.pallas import tpu_sc as plsc

_N = 10000
_E = 320000
_G = 16
_NC, _NS = 2, 16
_NW = _NC * _NS
_EPW = _E // _NW
_CH = 80
_NCHUNK = _EPW // _CH
_RPT = _N // _NS
_INV_SQRT_NN = 1.0 / math.sqrt(32.0)

_MESH = dict(core_axis_name="c", subcore_axis_name="s",
             num_cores=_NC, num_subcores=_NS)


_NBUF = 5
_NSUP = _NCHUNK // _NBUF


def _geom_scratch():
    refs = []
    for _ in range(_NBUF):
        refs += [
            pltpu.VMEM((_CH,), jnp.int32),
            pltpu.VMEM((_CH,), jnp.int32),
            pltpu.VMEM((_CH, 16), jnp.float32),
            pltpu.VMEM((_CH, 16), jnp.float32),
            pltpu.SemaphoreType.DMA,
        ]
    return refs


@functools.partial(
    pl.kernel,
    out_type=jax.ShapeDtypeStruct((_E, 16), jnp.float32),
    mesh=plsc.VectorSubcoreMesh(**_MESH),
    scratch_types=_geom_scratch(),
    compiler_params=pltpu.CompilerParams(use_tc_tiling_on_sc=False),
)
def _geom(pos_hbm, src_hbm, dst_hbm, vec_hbm, *scr):
    cid = lax.axis_index("c")
    sid = lax.axis_index("s")
    wid = cid * _NS + sid
    slots = [scr[5 * b:5 * b + 5] for b in range(_NBUF)]

    def super_it(g, carry):
        base = wid * _EPW + g * (_NBUF * _CH)
        di = []
        for b, (sidx, didx, ps, pd, sem) in enumerate(slots):
            off = base + b * _CH
            di.append((
                pltpu.async_copy(src_hbm.at[pl.ds(off, _CH)], sidx, sem),
                pltpu.async_copy(dst_hbm.at[pl.ds(off, _CH)], didx, sem),
            ))
        dg = []
        for b, (sidx, didx, ps, pd, sem) in enumerate(slots):
            di[b][0].wait()
            di[b][1].wait()
            dg.append((
                pltpu.async_copy(pos_hbm.at[sidx], ps, sem),
                pltpu.async_copy(pos_hbm.at[didx], pd, sem),
            ))
        dv = []
        for b, (sidx, didx, ps, pd, sem) in enumerate(slots):
            off = base + b * _CH
            dg[b][0].wait()
            dg[b][1].wait()

            @plsc.parallel_loop(0, _CH, unroll=4)
            def row(i, ps=ps, pd=pd):
                ps[i, :] = ps[i, :] - pd[i, :]

            dv.append(pltpu.async_copy(ps, vec_hbm.at[pl.ds(off, _CH)], sem))
        for d in dv:
            d.wait()
        return carry

    lax.fori_loop(0, _NSUP, super_it, 0)


@functools.lru_cache(maxsize=None)
def _make_msg(dp):
    nbuf = 2 if dp == 128 else 5
    nsup = _NCHUNK // nbuf
    ntail = _NCHUNK - nsup * nbuf
    scratch = []
    for _ in range(nbuf):
        scratch += [
            pltpu.VMEM((_CH,), jnp.int32),
            pltpu.VMEM((_CH,), jnp.int32),
            pltpu.VMEM((_CH, dp), jnp.float32),
            pltpu.VMEM((_CH, dp), jnp.float32),
            pltpu.SemaphoreType.DMA,
        ]
    scratch.append(pltpu.VMEM_SHARED((_N, dp), jnp.float32))

    @functools.partial(
        pl.kernel,
        out_type=jax.ShapeDtypeStruct((2 * _N, dp), jnp.float32),
        mesh=plsc.VectorSubcoreMesh(**_MESH),
        scratch_types=scratch,
        compiler_params=pltpu.CompilerParams(use_tc_tiling_on_sc=False),
    )
    def msg(x_hbm, c_hbm, src_hbm, dst_hbm, out_hbm, *scr):
        cid = lax.axis_index("c")
        sid = lax.axis_index("s")
        wid = cid * _NS + sid
        slots = [scr[5 * b:5 * b + 5] for b in range(nbuf)]
        aggsh = scr[-1]
        zv = jnp.zeros((16,), jnp.float32)

        xg0 = slots[0][2]

        @plsc.parallel_loop(0, _CH, unroll=4)
        def zrow(i):
            for j in range(dp // 16):
                xg0[i, pl.ds(j * 16, 16)] = zv

        nfull = _RPT // _CH
        rem = _RPT - nfull * _CH

        def zslab(t, c2):
            pltpu.sync_copy(xg0, aggsh.at[pl.ds(sid * _RPT + t * _CH, _CH)])
            return c2

        lax.fori_loop(0, nfull, zslab, 0)
        if rem:
            pltpu.sync_copy(xg0.at[pl.ds(0, rem)],
                            aggsh.at[pl.ds(sid * _RPT + nfull * _CH, rem)])
        plsc.subcore_barrier()

        def super_it(g, carry):
            base = wid * _EPW + g * (nbuf * _CH)
            di = []
            for b, (sidx, didx, xg, cg, sem) in enumerate(slots):
                off = base + b * _CH
                di.append((
                    pltpu.async_copy(src_hbm.at[pl.ds(off, _CH)], sidx, sem),
                    pltpu.async_copy(dst_hbm.at[pl.ds(off, _CH)], didx, sem),
                ))
            dg = []
            for b, (sidx, didx, xg, cg, sem) in enumerate(slots):
                off = base + b * _CH
                di[b][0].wait()
                di[b][1].wait()
                dg.append((
                    pltpu.async_copy(x_hbm.at[sidx], xg, sem),
                    pltpu.async_copy(c_hbm.at[pl.ds(off, _CH)], cg, sem),
                ))
            ds = []
            for b, (sidx, didx, xg, cg, sem) in enumerate(slots):
                dg[b][0].wait()
                dg[b][1].wait()

                @plsc.parallel_loop(0, _CH, unroll=2)
                def mrow(i, xg=xg, cg=cg):
                    for j in range(dp // 16):
                        sl = pl.ds(j * 16, 16)
                        xg[i, sl] = xg[i, sl] * cg[i, sl]

                ds.append(pltpu.async_copy(xg, aggsh.at[didx], sem, add=True))
            for d in ds:
                d.wait()
            return carry

        lax.fori_loop(0, nsup, super_it, 0)

        def tail(k, carry):
            off = wid * _EPW + (nsup * nbuf + k) * _CH
            sidx, didx, xg, cg, sem = slots[0]
            pltpu.sync_copy(src_hbm.at[pl.ds(off, _CH)], sidx)
            pltpu.sync_copy(dst_hbm.at[pl.ds(off, _CH)], didx)
            pltpu.async_copy(x_hbm.at[sidx], xg, sem).wait()
            pltpu.sync_copy(c_hbm.at[pl.ds(off, _CH)], cg)

            @plsc.parallel_loop(0, _CH, unroll=2)
            def mrow(i):
                for j in range(dp // 16):
                    sl = pl.ds(j * 16, 16)
                    xg[i, sl] = xg[i, sl] * cg[i, sl]

            pltpu.sync_copy(xg, aggsh.at[didx], add=True)
            return carry

        if ntail:
            lax.fori_loop(0, ntail, tail, 0)
        plsc.subcore_barrier()
        pltpu.sync_copy(aggsh.at[pl.ds(sid * _RPT, _RPT)],
                        out_hbm.at[pl.ds(cid * _N + sid * _RPT, _RPT)])

    return msg


_BE = 2000
_S3 = math.sqrt(3.0)
_S5 = math.sqrt(5.0)
_S15 = math.sqrt(15.0)
_RB_VALS = np.linspace(0.0, 5.0, 12)[1:-1]
_RB_STEP = float(_RB_VALS[1] - _RB_VALS[0])
_RB_C = 1.14136 * (math.e ** 2)
_SQRT_NB = math.sqrt(10.0)


def _coef_body(dps, *refs):
    vec_ref = refs[0]
    wrefs = refs[1:1 + 4 * len(dps)]
    orefs = refs[1 + 4 * len(dps):]

    v = vec_ref[...]
    x = v[:, 0:1]
    y = v[:, 1:2]
    z = v[:, 2:3]
    r = jnp.sqrt(x * x + y * y + z * z + 1e-9)
    inv = 1.0 / r
    ux, uy, uz = x * inv, y * inv, z * inv
    one = jnp.ones_like(ux)
    zero = jnp.zeros_like(ux)
    sh = jnp.concatenate([
        one,
        _S3 * ux, _S3 * uy, _S3 * uz,
        _S15 * ux * uz, _S15 * ux * uy,
        _S5 * (uy * uy - 0.5 * (ux * ux + uz * uz)),
        _S15 * uy * uz, 0.5 * _S15 * (uz * uz - ux * ux),
        zero, zero, zero, zero, zero, zero, zero,
    ], axis=1)

    kk = lax.broadcasted_iota(jnp.int32, (1, 10), 1).astype(jnp.float32)
    vals = (kk + 1.0) * _RB_STEP
    diff = (r - vals) * (1.0 / _RB_STEP)
    inside = jnp.abs(diff) < 1.0
    denom = jnp.maximum(jnp.where(inside, 1.0 - diff * diff, 1.0), 1e-6)
    emb = jnp.where(inside, _RB_C * jnp.exp(-1.0 / denom), 0.0) * _SQRT_NB
    emb = jnp.concatenate([emb, jnp.zeros_like(v[:, 0:6])], axis=1)

    for l in range(len(dps)):
        w1, b1, w2, wsh = wrefs[4 * l:4 * l + 4]
        h = jnp.dot(emb, w1[...], preferred_element_type=jnp.float32) + b1[...]
        h = h * jax.nn.sigmoid(h)
        c = (jnp.dot(h, w2[...], preferred_element_type=jnp.float32)
             * jnp.dot(sh, wsh[...], preferred_element_type=jnp.float32))
        orefs[l][...] = c


def _coef(vec, wlist, dps):
    flat_w = [w for tup in wlist for w in tup]
    full = lambda shape: pl.BlockSpec(shape, lambda i: (0, 0))
    return pl.pallas_call(
        functools.partial(_coef_body, tuple(dps)),
        grid=(_E // _BE,),
        in_specs=[pl.BlockSpec((_BE, 16), lambda i: (i, 0))]
        + [full(w.shape) for w in flat_w],
        out_specs=[pl.BlockSpec((_BE, dp), lambda i: (i, 0)) for dp in dps],
        out_shape=[jax.ShapeDtypeStruct((_E, dp), jnp.float32) for dp in dps],
    )(vec, *flat_w)


_BN = 2000


def _node_body(last, x_ref, na_ref, agg_ref, wsc_ref, wmsg_ref, o_ref):
    xn = x_ref[...] * na_ref[...]
    sc = jnp.dot(xn, wsc_ref[...], preferred_element_type=jnp.float32)
    a = (agg_ref[0] + agg_ref[1]) * _INV_SQRT_NN
    out = sc + jnp.dot(a, wmsg_ref[...], preferred_element_type=jnp.float32)
    if not last:
        out = out * jax.nn.sigmoid(out)
    o_ref[...] = out


def _node(xp, na, agg2, wsc, wmsg, last, dpo):
    dpi = xp.shape[1]
    agg3 = agg2.reshape(2, _N, dpi)
    return pl.pallas_call(
        functools.partial(_node_body, last),
        grid=(_N // _BN,),
        in_specs=[
            pl.BlockSpec((_BN, dpi), lambda i: (i, 0)),
            pl.BlockSpec((_BN, 1), lambda i: (i, 0)),
            pl.BlockSpec((2, _BN, dpi), lambda i: (0, i, 0)),
            pl.BlockSpec(wsc.shape, lambda i: (0, 0)),
            pl.BlockSpec(wmsg.shape, lambda i: (0, 0)),
        ],
        out_specs=pl.BlockSpec((_BN, dpo), lambda i: (i, 0)),
        out_shape=jax.ShapeDtypeStruct((_N, dpo), jnp.float32),
    )(xp, na, agg3, wsc, wmsg)


def _norm_body(x_ref, b_ref, o_ref):
    xo = x_ref[...]
    bt = b_ref[...]
    ss = jnp.sum(xo * xo, axis=1, keepdims=True)
    gids = lax.broadcasted_iota(jnp.int32, (1, _G), 1)
    oh = (bt == gids).astype(jnp.float32)
    g = jnp.sum(oh * ss, axis=0, keepdims=True)
    f = jnp.sqrt(g + 1e-12)
    fb = jnp.sum(oh * f, axis=1, keepdims=True)
    o_ref[...] = xo / fb


def _norm(xo, bt):
    return pl.pallas_call(
        _norm_body,
        in_specs=[
            pl.BlockSpec((_N, 128), lambda: (0, 0)),
            pl.BlockSpec((_N, 1), lambda: (0, 0)),
        ],
        out_specs=pl.BlockSpec((_N, 128), lambda: (0, 0)),
        out_shape=jax.ShapeDtypeStruct((_N, 128), jnp.float32),
    )(xo, bt)


def _padw(w, r, c):
    return jnp.zeros((r, c), jnp.float32).at[:w.shape[0], :w.shape[1]].set(w)


def _run_net(pos16, xin, na, src, dst, layers):
    vec = _geom(pos16, src, dst)
    dpis = [128, 64, 64, 64]
    dpos = [64, 64, 64, 128]
    wlist = [
        (
            _padw(p['W1'], 16, 64),
            p['b1'].reshape(1, 64),
            _padw(p['W2'], 64, dpis[l]),
            _padw(p['W_sh'], 16, dpis[l]),
        )
        for l, p in enumerate(layers)
    ]
    cs = _coef(vec, wlist, dpis)
    h = xin
    for l, p in enumerate(layers):
        agg2 = _make_msg(dpis[l])(h, cs[l], src, dst)
        wsc = _padw(p['W_sc'], dpis[l], dpos[l])
        wmsg = _padw(p['W_msg'], dpis[l], dpos[l])
        h = _node(h, na, agg2, wsc, wmsg, last=(l == 3), dpo=dpos[l])
    return h


def kernel(x, pos, batch, edge_index, node_attr, x_final_state,
           pos_final_state, edge_index_final_state, params):
    f32 = jnp.float32
    src = edge_index[0].astype(jnp.int32)
    dst = edge_index[1].astype(jnp.int32)
    srcf = edge_index_final_state[0].astype(jnp.int32)
    dstf = edge_index_final_state[1].astype(jnp.int32)
    pos16 = jnp.zeros((_N, 16), f32).at[:, :3].set(pos)
    posf16 = jnp.zeros((_N, 16), f32).at[:, :3].set(pos_final_state)
    post16 = 0.5 * (pos16 + posf16)
    bt = batch.astype(jnp.int32).reshape(_N, 1)

    na = node_attr
    out_i = _norm(_run_net(pos16, x, na, src, dst, params['init']), bt)
    out_f = _norm(_run_net(posf16, x_final_state, na, srcf, dstf,
                           params['final']), bt)
    x_ts = 0.5 * (out_i + out_f)
    out_ts = _norm(_run_net(post16, x_ts, na, src, dst, params['interp']), bt)
    return out_ts
